# baseline (speedup 1.0000x reference)
import functools

import jax
import jax.numpy as jnp
from jax.experimental import pallas as pl
from jax.experimental.pallas import tpu as pltpu
from jax.experimental.pallas import tpu_sc as plsc

_N_EXPERTS = 64
_TOPK = 2
_BLOCK_T = 4096
_LANES = 16


def _router_tc_kernel(x_ref, w_ref, logits_ref, weights_ref):
    x = x_ref[...]
    w = w_ref[...]
    logits = jax.lax.dot_general(
        x, w, (((1,), (1,)), ((), ())), preferred_element_type=jnp.float32
    )
    m = jnp.max(logits, axis=1, keepdims=True)
    e = jnp.exp(logits - m)
    s = jnp.sum(e, axis=1, keepdims=True)
    logits_ref[...] = logits
    weights_ref[...] = e / s


def _make_sc_topk(tokens, n_experts):
    info = plsc.get_sparse_core_info()
    n_workers = info.num_cores * info.num_subcores
    tpw = tokens // n_workers
    mesh = plsc.VectorSubcoreMesh(core_axis_name="c", subcore_axis_name="s")

    @functools.partial(
        pl.kernel,
        mesh=mesh,
        out_type=jax.ShapeDtypeStruct((tokens * _TOPK,), jnp.int32),
        scratch_types=[
            pltpu.VMEM((tpw * n_experts,), jnp.float32),
            pltpu.VMEM((tpw * _TOPK,), jnp.int32),
        ],
        compiler_params=pltpu.CompilerParams(needs_layout_passes=False),
    )
    def _sc_topk(logits_hbm, idx_hbm, ltile, itile):
        wid = jax.lax.axis_index("s") * info.num_cores + jax.lax.axis_index("c")
        base = wid * tpw
        pltpu.sync_copy(
            logits_hbm.at[pl.ds(base * n_experts, tpw * n_experts)], ltile
        )

        lane = jax.lax.broadcasted_iota(jnp.int32, (_LANES,), 0)
        zeros = jnp.zeros((_LANES,), jnp.int32)
        ninf = jnp.full((_LANES,), -jnp.inf, jnp.float32)

        def group_body(g, _):
            rows = lane + g * _LANES

            def expert_body(e, carry):
                m1, i1, m2, i2 = carry
                col = jnp.full((_LANES,), e, jnp.int32)
                val = plsc.load_gather(ltile, [rows * n_experts + col])
                gt1 = val > m1
                gt2 = val > m2
                i2 = jnp.where(gt1, i1, jnp.where(gt2, col, i2))
                m2 = jnp.where(gt1, m1, jnp.where(gt2, val, m2))
                i1 = jnp.where(gt1, col, i1)
                m1 = jnp.where(gt1, val, m1)
                return m1, i1, m2, i2

            m1, i1, m2, i2 = jax.lax.fori_loop(
                0, n_experts, expert_body, (ninf, zeros, ninf, zeros)
            )
            plsc.store_scatter(itile, [rows * _TOPK], i1)
            plsc.store_scatter(itile, [rows * _TOPK + 1], i2)
            return 0

        jax.lax.fori_loop(0, tpw // _LANES, group_body, 0)
        pltpu.sync_copy(itile, idx_hbm.at[pl.ds(base * _TOPK, tpw * _TOPK)])

    return _sc_topk


def kernel(hidden_states, W):
    tokens, hidden = hidden_states.shape
    n_experts = W.shape[0]
    bt = min(_BLOCK_T, tokens)
    grid = (tokens // bt,)
    out_shape = [
        jax.ShapeDtypeStruct((tokens, n_experts), jnp.float32),
        jax.ShapeDtypeStruct((tokens, n_experts), jnp.float32),
    ]
    logits, weights = pl.pallas_call(
        _router_tc_kernel,
        grid=grid,
        in_specs=[
            pl.BlockSpec((bt, hidden), lambda i: (i, 0)),
            pl.BlockSpec((n_experts, hidden), lambda i: (0, 0)),
        ],
        out_specs=[
            pl.BlockSpec((bt, n_experts), lambda i: (i, 0)),
            pl.BlockSpec((bt, n_experts), lambda i: (i, 0)),
        ],
        out_shape=out_shape,
        compiler_params=pltpu.CompilerParams(
            dimension_semantics=("parallel",),
        ),
    )(hidden_states, W)
    indices = _make_sc_topk(tokens, n_experts)(logits.reshape(-1))
    return (logits, weights, indices.reshape(tokens, _TOPK))


# speedup vs baseline: 1.1709x; 1.1709x over previous
import jax
import jax.numpy as jnp
from jax import lax
from jax.experimental import pallas as pl
from jax.experimental.pallas import tpu as pltpu
from jax.experimental.pallas import tpu_sc as plsc

_N_EXPERTS = 64
_TOPK = 2
_BLOCK_T = 4096
_L = 16


def _router_tc_kernel(x_ref, w_ref, logits_ref, weights_ref):
    x = x_ref[...]
    w = w_ref[...]
    logits = lax.dot_general(
        x, w, (((1,), (1,)), ((), ())), preferred_element_type=jnp.float32
    )
    m = jnp.max(logits, axis=1, keepdims=True)
    e = jnp.exp(logits - m)
    s = jnp.sum(e, axis=1, keepdims=True)
    logits_ref[...] = logits
    weights_ref[...] = e / s


def _sc_top2(logits, tokens, n_experts):
    info = plsc.get_sparse_core_info()
    nw = info.num_cores * info.num_subcores
    tpw = tokens // nw
    grp = 8
    nvr = n_experts // _L
    mesh = plsc.VectorSubcoreMesh(core_axis_name="c", subcore_axis_name="s")

    ch = 256

    def body(logits_hbm, idx_hbm, lv, iv):
        wid = lax.axis_index("s") * info.num_cores + lax.axis_index("c")
        base = wid * tpw
        lane = lax.broadcasted_iota(jnp.int32, (_L,), 0)
        neg = jnp.float32(-jnp.inf)
        big = jnp.int32(n_experts)

        def group(g, carry):
            acc = jnp.zeros((_L,), jnp.int32)
            for k in range(grp):
                t = g * grp + k
                v = [lv[t, pl.ds(j * _L, _L)] for j in range(nvr)]
                idx = [lane + j * _L for j in range(nvr)]
                vm = v[0]
                for j in range(1, nvr):
                    vm = jnp.maximum(vm, v[j])
                m1 = jnp.max(vm)
                cand = jnp.full((_L,), big, jnp.int32)
                for j in range(nvr):
                    cand = jnp.minimum(cand, jnp.where(v[j] == m1, idx[j], big))
                i1 = jnp.min(cand)
                vr = [jnp.where(idx[j] == i1, neg, v[j]) for j in range(nvr)]
                vm2 = vr[0]
                for j in range(1, nvr):
                    vm2 = jnp.maximum(vm2, vr[j])
                m2 = jnp.max(vm2)
                cand2 = jnp.full((_L,), big, jnp.int32)
                for j in range(nvr):
                    cand2 = jnp.minimum(cand2, jnp.where(vr[j] == m2, idx[j], big))
                i2 = jnp.min(cand2)
                acc = jnp.where(lane == 2 * k, i1, acc)
                acc = jnp.where(lane == 2 * k + 1, i2, acc)
            iv[pl.ds(g * 2 * grp, _L)] = acc
            return carry

        def chunk(c, carry):
            start = base + c * ch
            pltpu.sync_copy(logits_hbm.at[pl.ds(start, ch)], lv)
            lax.fori_loop(0, ch // grp, group, 0)
            pltpu.sync_copy(iv, idx_hbm.at[pl.ds(start * _TOPK, ch * _TOPK)])
            return carry

        lax.fori_loop(0, tpw // ch, chunk, 0)

    run = pl.kernel(
        body,
        out_type=jax.ShapeDtypeStruct((tokens * _TOPK,), jnp.int32),
        mesh=mesh,
        scratch_types=[
            pltpu.VMEM((ch, n_experts), jnp.float32),
            pltpu.VMEM((ch * _TOPK,), jnp.int32),
        ],
        compiler_params=pltpu.CompilerParams(needs_layout_passes=False),
    )
    return run(logits).reshape(tokens, _TOPK)


def kernel(hidden_states, W):
    tokens, hidden = hidden_states.shape
    n_experts = W.shape[0]
    bt = min(_BLOCK_T, tokens)
    grid = (tokens // bt,)
    out_shape = [
        jax.ShapeDtypeStruct((tokens, n_experts), jnp.float32),
        jax.ShapeDtypeStruct((tokens, n_experts), jnp.float32),
    ]
    logits, weights = pl.pallas_call(
        _router_tc_kernel,
        grid=grid,
        in_specs=[
            pl.BlockSpec((bt, hidden), lambda i: (i, 0)),
            pl.BlockSpec((n_experts, hidden), lambda i: (0, 0)),
        ],
        out_specs=[
            pl.BlockSpec((bt, n_experts), lambda i: (i, 0)),
            pl.BlockSpec((bt, n_experts), lambda i: (i, 0)),
        ],
        out_shape=out_shape,
        compiler_params=pltpu.CompilerParams(
            dimension_semantics=("parallel",),
        ),
    )(hidden_states, W)
    indices = _sc_top2(logits, tokens, n_experts)
    return (logits, weights, indices)


# speedup vs baseline: 1.9095x; 1.6307x over previous
import jax
import jax.numpy as jnp
from jax.experimental import pallas as pl
from jax.experimental.pallas import tpu as pltpu

_N_EXPERTS = 64
_TOPK = 2
_BLOCK_T = 4096


def _router_block_kernel(x_ref, w_ref, logits_ref, weights_ref, idx_ref):
    x = x_ref[...]
    w = w_ref[...]
    logits = jax.lax.dot_general(
        x, w, (((1,), (1,)), ((), ())), preferred_element_type=jnp.float32
    )
    m = jnp.max(logits, axis=1, keepdims=True)
    e = jnp.exp(logits - m)
    s = jnp.sum(e, axis=1, keepdims=True)
    logits_ref[...] = logits
    weights_ref[...] = e / s

    col = jax.lax.broadcasted_iota(jnp.int32, logits.shape, 1).astype(jnp.float32)
    big = jnp.float32(_N_EXPERTS)
    i1 = jnp.min(jnp.where(logits == m, col, big), axis=1, keepdims=True)
    rest = jnp.where(col == i1, -jnp.inf, logits)
    m2 = jnp.max(rest, axis=1, keepdims=True)
    i2 = jnp.min(jnp.where(rest == m2, col, big), axis=1, keepdims=True)
    idx_ref[...] = jnp.concatenate([i1, i2], axis=1).astype(jnp.int32)


def kernel(hidden_states, W):
    tokens, hidden = hidden_states.shape
    n_experts = W.shape[0]
    bt = min(_BLOCK_T, tokens)
    grid = (tokens // bt,)
    out_shape = [
        jax.ShapeDtypeStruct((tokens, n_experts), jnp.float32),
        jax.ShapeDtypeStruct((tokens, n_experts), jnp.float32),
        jax.ShapeDtypeStruct((tokens, _TOPK), jnp.int32),
    ]
    logits, weights, indices = pl.pallas_call(
        _router_block_kernel,
        grid=grid,
        in_specs=[
            pl.BlockSpec((bt, hidden), lambda i: (i, 0)),
            pl.BlockSpec((n_experts, hidden), lambda i: (0, 0)),
        ],
        out_specs=[
            pl.BlockSpec((bt, n_experts), lambda i: (i, 0)),
            pl.BlockSpec((bt, n_experts), lambda i: (i, 0)),
            pl.BlockSpec((bt, _TOPK), lambda i: (i, 0)),
        ],
        out_shape=out_shape,
        compiler_params=pltpu.CompilerParams(
            dimension_semantics=("parallel",),
        ),
    )(hidden_states, W)
    return (logits, weights, indices)
